# Initial kernel scaffold; baseline (speedup 1.0000x reference)
import jax
import jax.numpy as jnp
from jax.experimental import pallas as pl


def kernel(x, coords, idx, B_gauss, W1, b1, W2, b2, Wq, bq, Wk, bk, Wv, bv, Wo, bo):
    raise NotImplementedError("write your pallas kernel here")


# speedup vs baseline: 16.6074x; 16.6074x over previous
import functools
import math

import jax
import jax.numpy as jnp
from jax import lax
from jax.experimental import pallas as pl
from jax.experimental.pallas import tpu as pltpu
from jax.experimental.pallas import tpu_sc as plsc

B, C, N, K = 2, 128, 10000, 16
DIM_L = 128
F = DIM_L // 2
CP = 16
NK = N * K

NC, NS = 2, 16
NW = NC * NS
ROWS = B * NK
RPW = ROWS // NW
CH = 400
NCHUNK = RPW // CH

TN = 400
TNK = TN * K


def _trig_body(cpad, Bg, trig):
    p = jnp.dot(cpad[...], Bg[...], preferred_element_type=jnp.float32)
    ang = (2.0 * math.pi) * p
    trig[...] = jnp.concatenate([jnp.sin(ang), jnp.cos(ang)], axis=-1)


def _trig_table(cpad2d, Bg):
    return pl.pallas_call(
        _trig_body,
        out_shape=jax.ShapeDtypeStruct((B * N, DIM_L), jnp.float32),
    )(cpad2d, Bg)


def _sc_gather(table_f, table_t, idx_flat):
    mesh = plsc.VectorSubcoreMesh(core_axis_name="c", subcore_axis_name="s")

    @functools.partial(
        pl.kernel,
        out_type=(
            jax.ShapeDtypeStruct((B, NK, C), jnp.float32),
            jax.ShapeDtypeStruct((B, NK, DIM_L), jnp.float32),
        ),
        mesh=mesh,
        compiler_params=pltpu.CompilerParams(use_tc_tiling_on_sc=False),
        scratch_types=[
            pltpu.VMEM((CH,), jnp.int32),
            pltpu.VMEM((CH, C), jnp.float32),
            pltpu.VMEM((CH, DIM_L), jnp.float32),
            pltpu.SemaphoreType.DMA,
            pltpu.SemaphoreType.DMA,
        ],
    )
    def k(tf_hbm, tt_hbm, idx_hbm, feats_out, trig_out, idx_v, rows_f, rows_t,
          sem_f, sem_t):
        wid = lax.axis_index("s") * NC + lax.axis_index("c")
        b = wid // NS
        wl = wid % NS
        base = wl * RPW
        tbl_base = b * N

        def chunk(j, _):
            off = base + j * CH
            pltpu.sync_copy(idx_hbm.at[pl.ds(wid * RPW + j * CH, CH)], idx_v)
            for i in range(CH // 16):
                sl = pl.ds(i * 16, 16)
                idx_v[sl] = idx_v[sl] + tbl_base
            cp_f = pltpu.async_copy(tf_hbm.at[idx_v], rows_f, sem_f)
            cp_t = pltpu.async_copy(tt_hbm.at[idx_v], rows_t, sem_t)
            cp_f.wait()
            cp_t.wait()
            pltpu.sync_copy(rows_f, feats_out.at[b, pl.ds(off, CH)])
            pltpu.sync_copy(rows_t, trig_out.at[b, pl.ds(off, CH)])
            return _

        lax.fori_loop(0, NCHUNK, chunk, None)

    return k(table_f, table_t, idx_flat)


def _tc_body(feats_g, trig_g, xsT, trig_c, W1T, b1, W2T, b2,
             WqT, bq, WkT, bk, WvT, bv, WoT, bo, out):
    fg = feats_g[0]
    tg = trig_g[0]
    xc = xsT[0]
    tc = trig_c[0]

    sg = tg[:, :F].reshape(TN, K, F)
    cg = tg[:, F:].reshape(TN, K, F)
    sc = tc[:, :F][:, None, :]
    cc = tc[:, F:][:, None, :]
    ef = jnp.concatenate([sg * cc - cg * sc, cg * cc + sg * sc],
                         axis=-1).reshape(TNK, DIM_L)

    h = jnp.maximum(jnp.dot(ef, W1T[...],
                            preferred_element_type=jnp.float32) + b1[...], 0.0)
    eg = jnp.dot(h, W2T[...], preferred_element_type=jnp.float32) + b2[...]

    kv = fg + eg
    km = jnp.dot(kv, WkT[...], preferred_element_type=jnp.float32) + bk[...]
    vm = jnp.dot(kv, WvT[...], preferred_element_type=jnp.float32) + bv[...]
    q = jnp.dot(xc, WqT[...], preferred_element_type=jnp.float32) + bq[...]

    logits = jnp.sum(q[:, None, :] * km.reshape(TN, K, C), axis=-1)
    logits = logits * (1.0 / math.sqrt(C))
    m = jnp.max(logits, axis=-1, keepdims=True)
    e = jnp.exp(logits - m)
    attn = e / jnp.sum(e, axis=-1, keepdims=True)

    o = jnp.sum(attn[:, :, None] * vm.reshape(TN, K, C), axis=1)
    out[0] = jnp.dot(o, WoT[...], preferred_element_type=jnp.float32) + bo[...]


def _tc_attend(feats_g, trig_g, xsT, trig_c, W1T, b1, W2T, b2,
               WqT, bq, WkT, bk, WvT, bv, WoT, bo):
    full = lambda *shape: pl.BlockSpec(shape, lambda b, t: (0,) * len(shape))
    grid = (B, N // TN)
    return pl.pallas_call(
        _tc_body,
        grid=grid,
        in_specs=[
            pl.BlockSpec((1, TNK, C), lambda b, t: (b, t, 0)),
            pl.BlockSpec((1, TNK, DIM_L), lambda b, t: (b, t, 0)),
            pl.BlockSpec((1, TN, C), lambda b, t: (b, t, 0)),
            pl.BlockSpec((1, TN, DIM_L), lambda b, t: (b, t, 0)),
            full(DIM_L, DIM_L // 4),
            full(1, DIM_L // 4),
            full(DIM_L // 4, C),
            full(1, C),
            full(C, C), full(1, C),
            full(C, C), full(1, C),
            full(C, C), full(1, C),
            full(C, C), full(1, C),
        ],
        out_specs=pl.BlockSpec((1, TN, C), lambda b, t: (b, t, 0)),
        out_shape=jax.ShapeDtypeStruct((B, N, C), jnp.float32),
    )(feats_g, trig_g, xsT, trig_c, W1T, b1, W2T, b2,
      WqT, bq, WkT, bk, WvT, bv, WoT, bo)


def kernel(x, coords, idx, B_gauss, W1, b1, W2, b2, Wq, bq, Wk, bk, Wv, bv, Wo, bo):
    xsT = jnp.transpose(x[..., 0], (0, 2, 1))
    cpad2d = jnp.concatenate(
        [coords, jnp.zeros((B, N, CP - 4), jnp.float32)],
        axis=-1).reshape(B * N, CP)
    Bg = jnp.concatenate(
        [B_gauss, jnp.zeros((CP - 4, F), jnp.float32)], axis=0)
    idx_flat = idx.astype(jnp.int32).reshape(ROWS)

    trig = _trig_table(cpad2d, Bg)

    feats_g, trig_g = _sc_gather(xsT.reshape(B * N, C), trig, idx_flat)

    out_nc = _tc_attend(
        feats_g, trig_g, xsT, trig.reshape(B, N, DIM_L),
        W1.T, b1.reshape(1, -1), W2.T, b2.reshape(1, -1),
        Wq.T, bq.reshape(1, -1), Wk.T, bk.reshape(1, -1),
        Wv.T, bv.reshape(1, -1), Wo.T, bo.reshape(1, -1))

    return jnp.transpose(out_nc, (0, 2, 1))[..., None]


# speedup vs baseline: 17.2890x; 1.0410x over previous
import functools
import math

import jax
import jax.numpy as jnp
from jax import lax
from jax.experimental import pallas as pl
from jax.experimental.pallas import tpu as pltpu
from jax.experimental.pallas import tpu_sc as plsc

B, C, N, K = 2, 128, 10000, 16
DIM_L = 128
F = DIM_L // 2
CP = 16
NK = N * K

NC, NS = 2, 16
NW = NC * NS
ROWS = B * NK
RPW = ROWS // NW
CH = 200
NCHUNK = RPW // CH
NSUP = NCHUNK // 2

TN = 400
TNK = TN * K


def _trig_body(cpad, Bg, trig):
    p = jnp.dot(cpad[...], Bg[...], preferred_element_type=jnp.float32)
    ang = (2.0 * math.pi) * p
    trig[...] = jnp.concatenate([jnp.sin(ang), jnp.cos(ang)], axis=-1)


TRIG_G = 10


def _trig_table(cpad2d, Bg):
    rows = B * N // TRIG_G
    return pl.pallas_call(
        _trig_body,
        grid=(TRIG_G,),
        in_specs=[
            pl.BlockSpec((rows, CP), lambda i: (i, 0)),
            pl.BlockSpec((CP, F), lambda i: (0, 0)),
        ],
        out_specs=pl.BlockSpec((rows, DIM_L), lambda i: (i, 0)),
        out_shape=jax.ShapeDtypeStruct((B * N, DIM_L), jnp.float32),
    )(cpad2d, Bg)


def _sc_gather(table_f, table_t, idx_flat):
    mesh = plsc.VectorSubcoreMesh(core_axis_name="c", subcore_axis_name="s")

    @functools.partial(
        pl.kernel,
        out_type=(
            jax.ShapeDtypeStruct((B, NK, C), jnp.float32),
            jax.ShapeDtypeStruct((B, NK, DIM_L), jnp.float32),
        ),
        mesh=mesh,
        compiler_params=pltpu.CompilerParams(use_tc_tiling_on_sc=False),
        scratch_types=[
            pltpu.VMEM((CH,), jnp.int32),
            pltpu.VMEM((CH,), jnp.int32),
            pltpu.VMEM((CH, C), jnp.float32),
            pltpu.VMEM((CH, C), jnp.float32),
            pltpu.VMEM((CH, DIM_L), jnp.float32),
            pltpu.VMEM((CH, DIM_L), jnp.float32),
            pltpu.SemaphoreType.DMA,
            pltpu.SemaphoreType.DMA,
            pltpu.SemaphoreType.DMA,
            pltpu.SemaphoreType.DMA,
        ],
    )
    def k(tf_hbm, tt_hbm, idx_hbm, feats_out, trig_out,
          idx0, idx1, rf0, rf1, rt0, rt1, sf0, sf1, st0, st1):
        wid = lax.axis_index("s") * NC + lax.axis_index("c")
        b = wid // NS
        wl = wid % NS
        base = wl * RPW

        def fire(j, idx_v, rf, rt, sf, st):
            pltpu.sync_copy(idx_hbm.at[pl.ds(wid * RPW + j * CH, CH)], idx_v)
            pltpu.async_copy(tf_hbm.at[idx_v], rf, sf)
            pltpu.async_copy(tt_hbm.at[idx_v], rt, st)

        def drain_write(j, idx_v, rf, rt, sf, st):
            pltpu.make_async_copy(tf_hbm.at[idx_v], rf, sf).wait()
            pltpu.make_async_copy(tt_hbm.at[idx_v], rt, st).wait()
            off = base + j * CH
            pltpu.sync_copy(rf, feats_out.at[b, pl.ds(off, CH)])
            pltpu.sync_copy(rt, trig_out.at[b, pl.ds(off, CH)])

        fire(0, idx0, rf0, rt0, sf0, st0)

        def sup(s, _):
            fire(2 * s + 1, idx1, rf1, rt1, sf1, st1)
            drain_write(2 * s, idx0, rf0, rt0, sf0, st0)

            @pl.when(s < NSUP - 1)
            def _fire_next():
                fire(2 * s + 2, idx0, rf0, rt0, sf0, st0)

            drain_write(2 * s + 1, idx1, rf1, rt1, sf1, st1)
            return _

        lax.fori_loop(0, NSUP, sup, None)

    return k(table_f, table_t, idx_flat)


def _tc_body(feats_g, trig_g, xsT, trig_c, W1T, b1, W2T, b2,
             WqT, bq, WkT, bk, WvT, bv, WoT, bo, out):
    fg = feats_g[0]
    tg = trig_g[0]
    xc = xsT[0]
    tc = trig_c[0]

    sg = tg[:, :F].reshape(TN, K, F)
    cg = tg[:, F:].reshape(TN, K, F)
    sc = tc[:, :F][:, None, :]
    cc = tc[:, F:][:, None, :]
    ef = jnp.concatenate([sg * cc - cg * sc, cg * cc + sg * sc],
                         axis=-1).reshape(TNK, DIM_L)

    h = jnp.maximum(jnp.dot(ef, W1T[...],
                            preferred_element_type=jnp.float32) + b1[...], 0.0)
    eg = jnp.dot(h, W2T[...], preferred_element_type=jnp.float32) + b2[...]

    kv = fg + eg
    km = jnp.dot(kv, WkT[...], preferred_element_type=jnp.float32) + bk[...]
    vm = jnp.dot(kv, WvT[...], preferred_element_type=jnp.float32) + bv[...]
    q = jnp.dot(xc, WqT[...], preferred_element_type=jnp.float32) + bq[...]

    logits = jnp.sum(q[:, None, :] * km.reshape(TN, K, C), axis=-1)
    logits = logits * (1.0 / math.sqrt(C))
    m = jnp.max(logits, axis=-1, keepdims=True)
    e = jnp.exp(logits - m)
    attn = e / jnp.sum(e, axis=-1, keepdims=True)

    o = jnp.sum(attn[:, :, None] * vm.reshape(TN, K, C), axis=1)
    out[0] = jnp.dot(o, WoT[...], preferred_element_type=jnp.float32) + bo[...]


def _tc_attend(feats_g, trig_g, xsT, trig_c, W1T, b1, W2T, b2,
               WqT, bq, WkT, bk, WvT, bv, WoT, bo):
    full = lambda *shape: pl.BlockSpec(shape, lambda b, t: (0,) * len(shape))
    grid = (B, N // TN)
    return pl.pallas_call(
        _tc_body,
        grid=grid,
        in_specs=[
            pl.BlockSpec((1, TNK, C), lambda b, t: (b, t, 0)),
            pl.BlockSpec((1, TNK, DIM_L), lambda b, t: (b, t, 0)),
            pl.BlockSpec((1, TN, C), lambda b, t: (b, t, 0)),
            pl.BlockSpec((1, TN, DIM_L), lambda b, t: (b, t, 0)),
            full(DIM_L, DIM_L // 4),
            full(1, DIM_L // 4),
            full(DIM_L // 4, C),
            full(1, C),
            full(C, C), full(1, C),
            full(C, C), full(1, C),
            full(C, C), full(1, C),
            full(C, C), full(1, C),
        ],
        out_specs=pl.BlockSpec((1, TN, C), lambda b, t: (b, t, 0)),
        out_shape=jax.ShapeDtypeStruct((B, N, C), jnp.float32),
    )(feats_g, trig_g, xsT, trig_c, W1T, b1, W2T, b2,
      WqT, bq, WkT, bk, WvT, bv, WoT, bo)


def kernel(x, coords, idx, B_gauss, W1, b1, W2, b2, Wq, bq, Wk, bk, Wv, bv, Wo, bo):
    xsT = jnp.transpose(x[..., 0], (0, 2, 1))
    cpad2d = jnp.concatenate(
        [coords, jnp.zeros((B, N, CP - 4), jnp.float32)],
        axis=-1).reshape(B * N, CP)
    Bg = jnp.concatenate(
        [B_gauss, jnp.zeros((CP - 4, F), jnp.float32)], axis=0)
    idx_flat = (idx.astype(jnp.int32)
                + (jnp.arange(B, dtype=jnp.int32) * N)[:, None, None]
                ).reshape(ROWS)

    trig = _trig_table(cpad2d, Bg)

    feats_g, trig_g = _sc_gather(xsT.reshape(B * N, C), trig, idx_flat)

    out_nc = _tc_attend(
        feats_g, trig_g, xsT, trig.reshape(B, N, DIM_L),
        W1.T, b1.reshape(1, -1), W2.T, b2.reshape(1, -1),
        Wq.T, bq.reshape(1, -1), Wk.T, bk.reshape(1, -1),
        Wv.T, bv.reshape(1, -1), Wo.T, bo.reshape(1, -1))

    return jnp.transpose(out_nc, (0, 2, 1))[..., None]


# speedup vs baseline: 20.8631x; 1.2067x over previous
import functools
import math

import jax
import jax.numpy as jnp
from jax import lax
from jax.experimental import pallas as pl
from jax.experimental.pallas import tpu as pltpu
from jax.experimental.pallas import tpu_sc as plsc

B, C, N, K = 2, 128, 10000, 16
DIM_L = 128
F = DIM_L // 2
CP = 16
NK = N * K

NC, NS = 2, 16
NW = NC * NS
ROWS = B * NK
RPW = ROWS // NW
CH = 200
NCHUNK = RPW // CH
NSUP = NCHUNK // 2

TN = 400
TNK = TN * K

TRIG_G = 10

def _unpack(words_i32):
    u = lax.bitcast_convert_type(words_i32, jnp.uint32)
    lo = lax.bitcast_convert_type(u << 16, jnp.float32)
    hi = lax.bitcast_convert_type((u >> 16) << 16, jnp.float32)
    return lo, hi


def _pack_body(feats, cpad, Bg, packed):
    p = jnp.dot(cpad[...], Bg[...], preferred_element_type=jnp.float32)
    ang = (2.0 * math.pi) * p
    trig = jnp.concatenate([jnp.sin(ang), jnp.cos(ang)], axis=-1)
    tb = trig.astype(jnp.bfloat16).astype(jnp.float32)
    fb = feats[...].astype(jnp.bfloat16).astype(jnp.float32)
    tw = (lax.bitcast_convert_type(tb, jnp.uint32) >> 16) << 16
    fw = lax.bitcast_convert_type(fb, jnp.uint32) >> 16
    packed[...] = lax.bitcast_convert_type(tw | fw, jnp.int32)


def _pack_table(xsT2d, cpad2d, Bg):
    rows = B * N // TRIG_G
    return pl.pallas_call(
        _pack_body,
        grid=(TRIG_G,),
        in_specs=[
            pl.BlockSpec((rows, C), lambda i: (i, 0)),
            pl.BlockSpec((rows, CP), lambda i: (i, 0)),
            pl.BlockSpec((CP, F), lambda i: (0, 0)),
        ],
        out_specs=pl.BlockSpec((rows, DIM_L), lambda i: (i, 0)),
        out_shape=jax.ShapeDtypeStruct((B * N, DIM_L), jnp.int32),
    )(xsT2d, cpad2d, Bg)


def _sc_gather(table, idx_flat):
    mesh = plsc.VectorSubcoreMesh(core_axis_name="c", subcore_axis_name="s")

    @functools.partial(
        pl.kernel,
        out_type=jax.ShapeDtypeStruct((B, NK, DIM_L), jnp.int32),
        mesh=mesh,
        compiler_params=pltpu.CompilerParams(use_tc_tiling_on_sc=False),
        scratch_types=[
            pltpu.VMEM((CH,), jnp.int32),
            pltpu.VMEM((CH,), jnp.int32),
            pltpu.VMEM((CH, DIM_L), jnp.int32),
            pltpu.VMEM((CH, DIM_L), jnp.int32),
            pltpu.SemaphoreType.DMA,
            pltpu.SemaphoreType.DMA,
        ],
    )
    def k(tbl_hbm, idx_hbm, out, idx0, idx1, r0, r1, s0, s1):
        wid = lax.axis_index("s") * NC + lax.axis_index("c")
        b = wid // NS
        wl = wid % NS
        base = wl * RPW

        def fire(j, idx_v, r, s):
            pltpu.sync_copy(idx_hbm.at[pl.ds(wid * RPW + j * CH, CH)], idx_v)
            pltpu.async_copy(tbl_hbm.at[idx_v], r, s)

        def drain_write(j, idx_v, r, s):
            pltpu.make_async_copy(tbl_hbm.at[idx_v], r, s).wait()
            pltpu.sync_copy(r, out.at[b, pl.ds(base + j * CH, CH)])

        fire(0, idx0, r0, s0)

        def sup(s, _):
            fire(2 * s + 1, idx1, r1, s1)
            drain_write(2 * s, idx0, r0, s0)

            @pl.when(s < NSUP - 1)
            def _fire_next():
                fire(2 * s + 2, idx0, r0, s0)

            drain_write(2 * s + 1, idx1, r1, s1)
            return _

        lax.fori_loop(0, NSUP, sup, None)

    return k(table, idx_flat)


def _tc_body(gw, cw, W1T, b1, W2T, b2, WqT, bq, WkT, bk, WvT, bv, WoT, bo, out):
    fg, tg = _unpack(gw[0])
    xc, ct = _unpack(cw[0])

    ctr = jnp.roll(ct, F, axis=-1)
    tg3 = tg.reshape(TN, K, DIM_L)
    p1 = tg3 * ctr[:, None, :]
    p2 = tg3 * ct[:, None, :]
    p1s = jnp.roll(p1, F, axis=-1)
    p2s = jnp.roll(p2, F, axis=-1)
    lane = lax.broadcasted_iota(jnp.int32, (TN, K, DIM_L), 2)
    ef3 = jnp.where(lane < F, p1 - p1s, p2 + p2s)
    ef = ef3.reshape(TNK, DIM_L).astype(jnp.bfloat16)

    h = jnp.maximum(jnp.dot(ef, W1T[...],
                            preferred_element_type=jnp.float32) + b1[...], 0.0)
    eg = jnp.dot(h.astype(jnp.bfloat16), W2T[...],
                 preferred_element_type=jnp.float32) + b2[...]

    kv = (fg + eg).astype(jnp.bfloat16)
    km = jnp.dot(kv, WkT[...], preferred_element_type=jnp.float32) + bk[...]
    vm = jnp.dot(kv, WvT[...], preferred_element_type=jnp.float32) + bv[...]
    q = jnp.dot(xc.astype(jnp.bfloat16), WqT[...],
                preferred_element_type=jnp.float32) + bq[...]

    logits = jnp.sum(q[:, None, :] * km.reshape(TN, K, C), axis=-1)
    logits = logits * (1.0 / math.sqrt(C))
    m = jnp.max(logits, axis=-1, keepdims=True)
    e = jnp.exp(logits - m)
    attn = e / jnp.sum(e, axis=-1, keepdims=True)

    o = jnp.sum(attn[:, :, None] * vm.reshape(TN, K, C), axis=1)
    out[0] = jnp.dot(o.astype(jnp.bfloat16), WoT[...],
                     preferred_element_type=jnp.float32) + bo[...]


def _tc_attend(packed_g, packed_c, W1T, b1, W2T, b2,
               WqT, bq, WkT, bk, WvT, bv, WoT, bo):
    full = lambda *shape: pl.BlockSpec(shape, lambda b, t: (0,) * len(shape))
    grid = (B, N // TN)
    return pl.pallas_call(
        _tc_body,
        grid=grid,
        in_specs=[
            pl.BlockSpec((1, TNK, DIM_L), lambda b, t: (b, t, 0)),
            pl.BlockSpec((1, TN, DIM_L), lambda b, t: (b, t, 0)),
            full(DIM_L, DIM_L // 4),
            full(1, DIM_L // 4),
            full(DIM_L // 4, C),
            full(1, C),
            full(C, C), full(1, C),
            full(C, C), full(1, C),
            full(C, C), full(1, C),
            full(C, C), full(1, C),
        ],
        out_specs=pl.BlockSpec((1, TN, C), lambda b, t: (b, t, 0)),
        out_shape=jax.ShapeDtypeStruct((B, N, C), jnp.float32),
    )(packed_g, packed_c, W1T, b1, W2T, b2,
      WqT, bq, WkT, bk, WvT, bv, WoT, bo)


def kernel(x, coords, idx, B_gauss, W1, b1, W2, b2, Wq, bq, Wk, bk, Wv, bv, Wo, bo):
    xsT = jnp.transpose(x[..., 0], (0, 2, 1))
    cpad2d = jnp.concatenate(
        [coords, jnp.zeros((B, N, CP - 4), jnp.float32)],
        axis=-1).reshape(B * N, CP)
    Bg = jnp.concatenate(
        [B_gauss, jnp.zeros((CP - 4, F), jnp.float32)], axis=0)
    idx_flat = (idx.astype(jnp.int32)
                + (jnp.arange(B, dtype=jnp.int32) * N)[:, None, None]
                ).reshape(ROWS)

    packed = _pack_table(xsT.reshape(B * N, C), cpad2d, Bg)
    packed_g = _sc_gather(packed, idx_flat)

    bf = jnp.bfloat16
    out_nc = _tc_attend(
        packed_g, packed.reshape(B, N, DIM_L),
        W1.T.astype(bf), b1.reshape(1, -1), W2.T.astype(bf), b2.reshape(1, -1),
        Wq.T.astype(bf), bq.reshape(1, -1), Wk.T.astype(bf), bk.reshape(1, -1),
        Wv.T.astype(bf), bv.reshape(1, -1), Wo.T.astype(bf), bo.reshape(1, -1))

    return jnp.transpose(out_nc, (0, 2, 1))[..., None]


# speedup vs baseline: 22.7292x; 1.0894x over previous
import functools
import math

import jax
import jax.numpy as jnp
from jax import lax
from jax.experimental import pallas as pl
from jax.experimental.pallas import tpu as pltpu
from jax.experimental.pallas import tpu_sc as plsc

B, C, N, K = 2, 128, 10000, 16
DIM_L = 128
F = DIM_L // 2
CP = 16
NK = N * K

NC, NS = 2, 16
NW = NC * NS
ROWS = B * NK
RPW = ROWS // NW
CH = 200
NCHUNK = RPW // CH
NSUP = NCHUNK // 2

TN = 400
TNK = TN * K

TRIG_G = 10

def _unpack(words_i32):
    u = lax.bitcast_convert_type(words_i32, jnp.uint32)
    lo = lax.bitcast_convert_type(u << 16, jnp.float32)
    hi = lax.bitcast_convert_type((u >> 16) << 16, jnp.float32)
    return lo, hi


def _pack_body(feats, cpad, Bg, packed):
    p = jnp.dot(cpad[...], Bg[...], preferred_element_type=jnp.float32)
    ang = (2.0 * math.pi) * p
    trig = jnp.concatenate([jnp.sin(ang), jnp.cos(ang)], axis=-1)
    tb = trig.astype(jnp.bfloat16).astype(jnp.float32)
    fb = feats[...].astype(jnp.bfloat16).astype(jnp.float32)
    tw = (lax.bitcast_convert_type(tb, jnp.uint32) >> 16) << 16
    fw = lax.bitcast_convert_type(fb, jnp.uint32) >> 16
    packed[...] = lax.bitcast_convert_type(tw | fw, jnp.int32)


def _pack_table(xsT2d, cpad2d, Bg):
    rows = B * N // TRIG_G
    return pl.pallas_call(
        _pack_body,
        grid=(TRIG_G,),
        in_specs=[
            pl.BlockSpec((rows, C), lambda i: (i, 0)),
            pl.BlockSpec((rows, CP), lambda i: (i, 0)),
            pl.BlockSpec((CP, F), lambda i: (0, 0)),
        ],
        out_specs=pl.BlockSpec((rows, DIM_L), lambda i: (i, 0)),
        out_shape=jax.ShapeDtypeStruct((B * N, DIM_L), jnp.int32),
    )(xsT2d, cpad2d, Bg)


RPW2 = NK // NW
NCHUNK2 = RPW2 // CH


def _sc_gather(table, idx_b):
    mesh = plsc.VectorSubcoreMesh(core_axis_name="c", subcore_axis_name="s")

    @functools.partial(
        pl.kernel,
        out_type=jax.ShapeDtypeStruct((NK, DIM_L), jnp.int32),
        mesh=mesh,
        compiler_params=pltpu.CompilerParams(use_tc_tiling_on_sc=False),
        scratch_types=[
            pltpu.VMEM((CH,), jnp.int32),
            pltpu.VMEM((CH,), jnp.int32),
            pltpu.VMEM((CH, DIM_L), jnp.int32),
            pltpu.VMEM((CH, DIM_L), jnp.int32),
            pltpu.SemaphoreType.DMA,
            pltpu.SemaphoreType.DMA,
        ],
    )
    def k(tbl_hbm, idx_hbm, out, idx0, idx1, r0, r1, s0, s1):
        wid = lax.axis_index("s") * NC + lax.axis_index("c")
        base = wid * RPW2

        def fire(j, idx_v, r, s):
            pltpu.sync_copy(idx_hbm.at[pl.ds(base + j * CH, CH)], idx_v)
            pltpu.async_copy(tbl_hbm.at[idx_v], r, s)

        def drain_write(j, idx_v, r, s):
            pltpu.make_async_copy(tbl_hbm.at[idx_v], r, s).wait()
            pltpu.sync_copy(r, out.at[pl.ds(base + j * CH, CH)])

        fire(0, idx0, r0, s0)

        def sup(s, _):
            fire(2 * s + 1, idx1, r1, s1)
            drain_write(2 * s, idx0, r0, s0)
            fire(2 * s + 2, idx0, r0, s0)
            drain_write(2 * s + 1, idx1, r1, s1)
            return _

        lax.fori_loop(0, NCHUNK2 // 2, sup, None)
        drain_write(NCHUNK2 - 1, idx0, r0, s0)

    return k(table, idx_b)


def _tc_body(gw, cw, W1T, b1, W2T, b2, WqT, bq, WkT, bk, WvT, bv, WoT, bo, out):
    fg, tg = _unpack(gw[...])
    xc, ct = _unpack(cw[...])

    ctr = jnp.roll(ct, F, axis=-1)
    tg3 = tg.reshape(TN, K, DIM_L)
    p1 = tg3 * ctr[:, None, :]
    p2 = tg3 * ct[:, None, :]
    p1s = jnp.roll(p1, F, axis=-1)
    p2s = jnp.roll(p2, F, axis=-1)
    lane = lax.broadcasted_iota(jnp.int32, (TN, K, DIM_L), 2)
    ef3 = jnp.where(lane < F, p1 - p1s, p2 + p2s)
    ef = ef3.reshape(TNK, DIM_L).astype(jnp.bfloat16)

    h = jnp.maximum(jnp.dot(ef, W1T[...],
                            preferred_element_type=jnp.float32) + b1[...], 0.0)
    eg = jnp.dot(h.astype(jnp.bfloat16), W2T[...],
                 preferred_element_type=jnp.float32) + b2[...]

    kv = (fg + eg).astype(jnp.bfloat16)
    km = jnp.dot(kv, WkT[...], preferred_element_type=jnp.float32) + bk[...]
    vm = jnp.dot(kv, WvT[...], preferred_element_type=jnp.float32) + bv[...]
    q = jnp.dot(xc.astype(jnp.bfloat16), WqT[...],
                preferred_element_type=jnp.float32) + bq[...]

    logits = jnp.sum(q[:, None, :] * km.reshape(TN, K, C), axis=-1)
    logits = logits * (1.0 / math.sqrt(C))
    m = jnp.max(logits, axis=-1, keepdims=True)
    e = jnp.exp(logits - m)
    attn = e / jnp.sum(e, axis=-1, keepdims=True)

    o = jnp.sum(attn[:, :, None] * vm.reshape(TN, K, C), axis=1)
    out[...] = jnp.dot(o.astype(jnp.bfloat16), WoT[...],
                       preferred_element_type=jnp.float32) + bo[...]


def _tc_attend(packed_g, packed_c, W1T, b1, W2T, b2,
               WqT, bq, WkT, bk, WvT, bv, WoT, bo):
    full = lambda *shape: pl.BlockSpec(shape, lambda t: (0,) * len(shape))
    return pl.pallas_call(
        _tc_body,
        grid=(N // TN,),
        in_specs=[
            pl.BlockSpec((TNK, DIM_L), lambda t: (t, 0)),
            pl.BlockSpec((TN, DIM_L), lambda t: (t, 0)),
            full(DIM_L, DIM_L // 4),
            full(1, DIM_L // 4),
            full(DIM_L // 4, C),
            full(1, C),
            full(C, C), full(1, C),
            full(C, C), full(1, C),
            full(C, C), full(1, C),
            full(C, C), full(1, C),
        ],
        out_specs=pl.BlockSpec((TN, C), lambda t: (t, 0)),
        out_shape=jax.ShapeDtypeStruct((N, C), jnp.float32),
    )(packed_g, packed_c, W1T, b1, W2T, b2,
      WqT, bq, WkT, bk, WvT, bv, WoT, bo)


def kernel(x, coords, idx, B_gauss, W1, b1, W2, b2, Wq, bq, Wk, bk, Wv, bv, Wo, bo):
    xsT = jnp.transpose(x[..., 0], (0, 2, 1))
    cpad2d = jnp.concatenate(
        [coords, jnp.zeros((B, N, CP - 4), jnp.float32)],
        axis=-1).reshape(B * N, CP)
    Bg = jnp.concatenate(
        [B_gauss, jnp.zeros((CP - 4, F), jnp.float32)], axis=0)
    idx_flat = (idx.astype(jnp.int32)
                + (jnp.arange(B, dtype=jnp.int32) * N)[:, None, None]
                ).reshape(ROWS)

    packed = _pack_table(xsT.reshape(B * N, C), cpad2d, Bg)
    packed3 = packed.reshape(B, N, DIM_L)
    idx2d = idx_flat.reshape(B, NK)

    bf = jnp.bfloat16
    weights = (
        W1.T.astype(bf), b1.reshape(1, -1), W2.T.astype(bf), b2.reshape(1, -1),
        Wq.T.astype(bf), bq.reshape(1, -1), Wk.T.astype(bf), bk.reshape(1, -1),
        Wv.T.astype(bf), bv.reshape(1, -1), Wo.T.astype(bf), bo.reshape(1, -1))

    out_nc = jnp.stack([
        _tc_attend(_sc_gather(packed, idx2d[b]), packed3[b], *weights)
        for b in range(B)])

    return jnp.transpose(out_nc, (0, 2, 1))[..., None]


# speedup vs baseline: 24.0001x; 1.0559x over previous
import functools
import math

import jax
import jax.numpy as jnp
from jax import lax
from jax.experimental import pallas as pl
from jax.experimental.pallas import tpu as pltpu
from jax.experimental.pallas import tpu_sc as plsc

B, C, N, K = 2, 128, 10000, 16
DIM_L = 128
F = DIM_L // 2
CP = 16
NK = N * K

NC, NS = 2, 16
NW = NC * NS
ROWS = B * NK
RPW = ROWS // NW
CH = 200
NCHUNK = RPW // CH
NSUP = NCHUNK // 2

TN = 400
TNK = TN * K

TRIG_G = 10

def _unpack(words_i32):
    u = lax.bitcast_convert_type(words_i32, jnp.uint32)
    lo = lax.bitcast_convert_type(u << 16, jnp.float32)
    hi = lax.bitcast_convert_type((u >> 16) << 16, jnp.float32)
    return lo, hi


def _pack_body(feats, cpad, Bg, packed):
    p = jnp.dot(cpad[...], Bg[...], preferred_element_type=jnp.float32)
    ang = (2.0 * math.pi) * p
    trig = jnp.concatenate([jnp.sin(ang), jnp.cos(ang)], axis=-1)
    tb = trig.astype(jnp.bfloat16).astype(jnp.float32)
    fb = feats[...].astype(jnp.bfloat16).astype(jnp.float32)
    tw = (lax.bitcast_convert_type(tb, jnp.uint32) >> 16) << 16
    fw = lax.bitcast_convert_type(fb, jnp.uint32) >> 16
    packed[...] = lax.bitcast_convert_type(tw | fw, jnp.int32)


def _pack_table(xsT2d, cpad2d, Bg):
    rows = B * N // TRIG_G
    return pl.pallas_call(
        _pack_body,
        grid=(TRIG_G,),
        in_specs=[
            pl.BlockSpec((rows, C), lambda i: (i, 0)),
            pl.BlockSpec((rows, CP), lambda i: (i, 0)),
            pl.BlockSpec((CP, F), lambda i: (0, 0)),
        ],
        out_specs=pl.BlockSpec((rows, DIM_L), lambda i: (i, 0)),
        out_shape=jax.ShapeDtypeStruct((B * N, DIM_L), jnp.int32),
    )(xsT2d, cpad2d, Bg)


RPW2 = NK // NW
NCHUNK2 = RPW2 // CH


def _sc_gather(table, idx_b):
    mesh = plsc.VectorSubcoreMesh(core_axis_name="c", subcore_axis_name="s")

    @functools.partial(
        pl.kernel,
        out_type=jax.ShapeDtypeStruct((NK, DIM_L), jnp.int32),
        mesh=mesh,
        compiler_params=pltpu.CompilerParams(use_tc_tiling_on_sc=False),
        scratch_types=[
            pltpu.VMEM((CH,), jnp.int32),
            pltpu.VMEM((CH,), jnp.int32),
            pltpu.VMEM((CH, DIM_L), jnp.int32),
            pltpu.VMEM((CH, DIM_L), jnp.int32),
            pltpu.SemaphoreType.DMA,
            pltpu.SemaphoreType.DMA,
        ],
    )
    def k(tbl_hbm, idx_hbm, out, idx0, idx1, r0, r1, s0, s1):
        wid = lax.axis_index("s") * NC + lax.axis_index("c")
        base = wid * RPW2

        def fire(j, idx_v, r, s):
            pltpu.sync_copy(idx_hbm.at[pl.ds(base + j * CH, CH)], idx_v)
            pltpu.async_copy(tbl_hbm.at[idx_v], r, s)

        def drain_write(j, idx_v, r, s):
            pltpu.make_async_copy(tbl_hbm.at[idx_v], r, s).wait()
            pltpu.sync_copy(r, out.at[pl.ds(base + j * CH, CH)])

        fire(0, idx0, r0, s0)

        def sup(s, _):
            fire(2 * s + 1, idx1, r1, s1)
            drain_write(2 * s, idx0, r0, s0)
            fire(2 * s + 2, idx0, r0, s0)
            drain_write(2 * s + 1, idx1, r1, s1)
            return _

        lax.fori_loop(0, NCHUNK2 // 2, sup, None)
        drain_write(NCHUNK2 - 1, idx0, r0, s0)

    return k(table, idx_b)


def _tc_body(gw, cw, W1T, b1, W2T, b2, WqT, bq, WkT, bk, WvT, bv, WoT, bo, out):
    fg, tg = _unpack(gw[...])
    xc, ct = _unpack(cw[...])

    ctr = jnp.roll(ct, F, axis=-1)
    tg3 = tg.reshape(TN, K, DIM_L)
    p1 = tg3 * ctr[:, None, :]
    p2 = tg3 * ct[:, None, :]
    p1s = jnp.roll(p1, F, axis=-1)
    p2s = jnp.roll(p2, F, axis=-1)
    lane = lax.broadcasted_iota(jnp.int32, (TN, K, DIM_L), 2)
    ef3 = jnp.where(lane < F, p1 - p1s, p2 + p2s)
    ef = ef3.reshape(TNK, DIM_L).astype(jnp.bfloat16)

    h = jnp.maximum(jnp.dot(ef, W1T[...],
                            preferred_element_type=jnp.float32) + b1[...], 0.0)
    eg = jnp.dot(h.astype(jnp.bfloat16), W2T[...],
                 preferred_element_type=jnp.float32) + b2[...]

    kv = (fg + eg).astype(jnp.bfloat16)
    km = jnp.dot(kv, WkT[...], preferred_element_type=jnp.float32) + bk[...]
    vm = jnp.dot(kv, WvT[...], preferred_element_type=jnp.float32) + bv[...]
    q = jnp.dot(xc.astype(jnp.bfloat16), WqT[...],
                preferred_element_type=jnp.float32) + bq[...]

    logits = jnp.sum(q[:, None, :] * km.reshape(TN, K, C), axis=-1)
    m = jnp.max(logits, axis=-1, keepdims=True)
    e = jnp.exp(logits - m)
    inv = 1.0 / jnp.sum(e, axis=-1, keepdims=True)

    o = jnp.sum(e[:, :, None] * vm.reshape(TN, K, C), axis=1) * inv
    out[...] = jnp.dot(o.astype(jnp.bfloat16), WoT[...],
                       preferred_element_type=jnp.float32) + bo[...]


def _tc_attend(packed_g, packed_c, W1T, b1, W2T, b2,
               WqT, bq, WkT, bk, WvT, bv, WoT, bo):
    full = lambda *shape: pl.BlockSpec(shape, lambda t: (0,) * len(shape))
    return pl.pallas_call(
        _tc_body,
        grid=(N // TN,),
        in_specs=[
            pl.BlockSpec((TNK, DIM_L), lambda t: (t, 0)),
            pl.BlockSpec((TN, DIM_L), lambda t: (t, 0)),
            full(DIM_L, DIM_L // 4),
            full(1, DIM_L // 4),
            full(DIM_L // 4, C),
            full(1, C),
            full(C, C), full(1, C),
            full(C, C), full(1, C),
            full(C, C), full(1, C),
            full(C, C), full(1, C),
        ],
        out_specs=pl.BlockSpec((TN, C), lambda t: (t, 0)),
        out_shape=jax.ShapeDtypeStruct((N, C), jnp.float32),
    )(packed_g, packed_c, W1T, b1, W2T, b2,
      WqT, bq, WkT, bk, WvT, bv, WoT, bo)


def kernel(x, coords, idx, B_gauss, W1, b1, W2, b2, Wq, bq, Wk, bk, Wv, bv, Wo, bo):
    xsT = jnp.transpose(x[..., 0], (0, 2, 1))
    cpad2d = jnp.concatenate(
        [coords, jnp.zeros((B, N, CP - 4), jnp.float32)],
        axis=-1).reshape(B * N, CP)
    Bg = jnp.concatenate(
        [B_gauss, jnp.zeros((CP - 4, F), jnp.float32)], axis=0)
    idx_flat = (idx.astype(jnp.int32)
                + (jnp.arange(B, dtype=jnp.int32) * N)[:, None, None]
                ).reshape(ROWS)

    packed = _pack_table(xsT.reshape(B * N, C), cpad2d, Bg)
    packed3 = packed.reshape(B, N, DIM_L)
    idx2d = idx_flat.reshape(B, NK)

    bf = jnp.bfloat16
    weights = (
        W1.T.astype(bf), b1.reshape(1, -1), W2.T.astype(bf), b2.reshape(1, -1),
        (Wq.T / math.sqrt(C)).astype(bf), bq.reshape(1, -1) / math.sqrt(C),
        Wk.T.astype(bf), bk.reshape(1, -1),
        Wv.T.astype(bf), bv.reshape(1, -1), Wo.T.astype(bf), bo.reshape(1, -1))

    out_nc = jnp.stack([
        _tc_attend(_sc_gather(packed, idx2d[b]), packed3[b], *weights)
        for b in range(B)])

    return jnp.transpose(out_nc, (0, 2, 1))[..., None]
